# baseline (speedup 1.0000x reference)
import functools

import jax
import jax.numpy as jnp
from jax import lax
from jax.experimental import pallas as pl
from jax.experimental.pallas import tpu as pltpu
from jax.experimental.pallas import tpu_sc as plsc

NUM_EXPERTS = 8
TOP_K = 2
HIDDEN = 1024
INTER = 4096
TOKENS = 2048

ASSIGN = TOKENS * TOP_K
BM = 256
NB = ASSIGN // BM + NUM_EXPERTS
P = NB * BM
BI = 512
NI = INTER // BI

_NC = 2
_NS = 16
_NW = _NC * _NS

_NEG = -1e30


def _router_body(x_ref, gw_ref, wts_ref, idx_ref):
    x = x_ref[...]
    gw = gw_ref[...]
    logits = lax.dot_general(x, gw, (((1,), (1,)), ((), ())),
                             preferred_element_type=jnp.float32)
    col = lax.broadcasted_iota(jnp.int32, logits.shape, 1)
    logits = jnp.where(col < NUM_EXPERTS, logits, _NEG)
    m1 = jnp.max(logits, axis=1, keepdims=True)
    i1 = jnp.min(jnp.where(logits == m1, col, 128), axis=1, keepdims=True)
    l2 = jnp.where(col == i1, _NEG, logits)
    m2 = jnp.max(l2, axis=1, keepdims=True)
    i2 = jnp.min(jnp.where(l2 == m2, col, 128), axis=1, keepdims=True)
    wa = 1.0 / (1.0 + jnp.exp(m2 - m1))
    wb = 1.0 - wa
    wts_ref[...] = jnp.where(col == 0, wa, 0.0) + jnp.where(col == 1, wb, 0.0)
    idx_ref[...] = jnp.where(col == 0, i1, 0) + jnp.where(col == 1, i2, 0)


def _router(x, gate_w):
    gw_pad = jnp.zeros((128, HIDDEN), jnp.float32).at[:NUM_EXPERTS].set(gate_w)
    wts128, idx128 = pl.pallas_call(
        _router_body,
        out_shape=(jax.ShapeDtypeStruct((TOKENS, 128), jnp.float32),
                   jax.ShapeDtypeStruct((TOKENS, 128), jnp.int32)),
    )(x, gw_pad)
    return wts128[:, :TOP_K], idx128[:, :TOP_K]


_T2 = TOKENS // _NW


@functools.lru_cache(maxsize=None)
def _sc_dispatch_kernel():
    mesh = plsc.VectorSubcoreMesh(core_axis_name="c", subcore_axis_name="s",
                                  num_cores=_NC, num_subcores=_NS)

    @functools.partial(
        pl.kernel, mesh=mesh,
        out_type=jax.ShapeDtypeStruct((P, HIDDEN), jnp.float32),
        scratch_types=[
            pltpu.VMEM((_T2,), jnp.int32),
            pltpu.VMEM((_T2,), jnp.int32),
            pltpu.VMEM((_T2, HIDDEN), jnp.float32),
            pltpu.SemaphoreType.DMA,
            pltpu.SemaphoreType.DMA,
            pltpu.SemaphoreType.DMA,
        ],
    )
    def k(x_hbm, p0_hbm, p1_hbm, xs_hbm, i0, i1, buf, sem0, sem1, sem2):
        wid = lax.axis_index("s") * _NC + lax.axis_index("c")
        base = wid * _T2
        cx = pltpu.async_copy(x_hbm.at[pl.ds(base, _T2)], buf, sem0)
        pltpu.sync_copy(p0_hbm.at[pl.ds(base, _T2)], i0)
        pltpu.sync_copy(p1_hbm.at[pl.ds(base, _T2)], i1)
        cx.wait()
        ca = pltpu.async_copy(buf, xs_hbm.at[i0], sem1)
        cb = pltpu.async_copy(buf, xs_hbm.at[i1], sem2)
        ca.wait()
        cb.wait()

    return k


def _sc_dispatch(x, pos0, pos1):
    return _sc_dispatch_kernel()(x, pos0, pos1)


_T4 = TOKENS // _NW
_G4 = 32


@functools.lru_cache(maxsize=None)
def _sc_combine_kernel():
    mesh = plsc.VectorSubcoreMesh(core_axis_name="c", subcore_axis_name="s",
                                  num_cores=_NC, num_subcores=_NS)

    @functools.partial(
        pl.kernel, mesh=mesh,
        out_type=jax.ShapeDtypeStruct((TOKENS, HIDDEN), jnp.float32),
        scratch_types=[
            pltpu.VMEM((_T4,), jnp.int32),
            pltpu.VMEM((_T4,), jnp.int32),
            pltpu.VMEM((_T4, 16), jnp.float32),
            pltpu.VMEM((_T4, 16), jnp.float32),
            pltpu.VMEM((_G4, HIDDEN), jnp.float32),
            pltpu.VMEM((_G4, HIDDEN), jnp.float32),
            pltpu.SemaphoreType.DMA,
            pltpu.SemaphoreType.DMA,
        ],
    )
    def k(ys_hbm, p0_hbm, p1_hbm, w0_hbm, w1_hbm, out_hbm,
          i0, i1, bw0, bw1, bufa, bufb, sema, semb):
        wid = lax.axis_index("s") * _NC + lax.axis_index("c")
        base = wid * _T4
        pltpu.sync_copy(p0_hbm.at[pl.ds(base, _T4)], i0)
        pltpu.sync_copy(p1_hbm.at[pl.ds(base, _T4)], i1)
        pltpu.sync_copy(w0_hbm.at[pl.ds(base, _T4)], bw0)
        pltpu.sync_copy(w1_hbm.at[pl.ds(base, _T4)], bw1)
        for g in range(_T4 // _G4):
            ca = pltpu.async_copy(
                ys_hbm.at[i0.at[pl.ds(g * _G4, _G4)]], bufa, sema)
            cb = pltpu.async_copy(
                ys_hbm.at[i1.at[pl.ds(g * _G4, _G4)]], bufb, semb)
            ca.wait()
            cb.wait()

            def add_row(r, _):
                wa = bw0[g * _G4 + r, pl.ds(0, 16)]
                wb = bw1[g * _G4 + r, pl.ds(0, 16)]
                for c in range(HIDDEN // 16):
                    sl = pl.ds(c * 16, 16)
                    bufa[r, sl] = bufa[r, sl] * wa + bufb[r, sl] * wb
                return _

            lax.fori_loop(0, _G4, add_row, 0)
            pltpu.sync_copy(bufa, out_hbm.at[pl.ds(base + g * _G4, _G4)])

    return k


def _sc_combine(ys, pos0, pos1, w0x, w1x):
    return _sc_combine_kernel()(ys, pos0, pos1, w0x, w1x)


def _gffn_body(be_ref, bv_ref, xs_ref, w1_ref, w3_ref, w2_ref, ys_ref):
    j = pl.program_id(0)
    i = pl.program_id(1)

    @pl.when(bv_ref[i] == 1)
    def _():
        xs = xs_ref[...]
        a1 = lax.dot_general(xs, w1_ref[0], (((1,), (1,)), ((), ())),
                             preferred_element_type=jnp.float32)
        a3 = lax.dot_general(xs, w3_ref[0], (((1,), (1,)), ((), ())),
                             preferred_element_type=jnp.float32)
        h = a1 * jax.nn.sigmoid(a1) * a3
        part = lax.dot_general(h, w2_ref[0], (((1,), (1,)), ((), ())),
                               preferred_element_type=jnp.float32)
        rows = pl.ds(i * BM, BM)

        @pl.when(j == 0)
        def _init():
            ys_ref[rows, :] = part

        @pl.when(j > 0)
        def _acc():
            ys_ref[rows, :] += part


def _grouped_ffn(xs, w1, w3, w2, be, bvalid):
    grid_spec = pltpu.PrefetchScalarGridSpec(
        num_scalar_prefetch=2,
        grid=(NI, NB),
        in_specs=[
            pl.BlockSpec((BM, HIDDEN), lambda j, i, be, bv: (i, 0)),
            pl.BlockSpec((1, BI, HIDDEN), lambda j, i, be, bv: (be[i], j, 0)),
            pl.BlockSpec((1, BI, HIDDEN), lambda j, i, be, bv: (be[i], j, 0)),
            pl.BlockSpec((1, HIDDEN, BI), lambda j, i, be, bv: (be[i], 0, j)),
        ],
        out_specs=pl.BlockSpec((P, HIDDEN), lambda j, i, be, bv: (0, 0)),
    )
    return pl.pallas_call(
        _gffn_body,
        grid_spec=grid_spec,
        out_shape=jax.ShapeDtypeStruct((P, HIDDEN), jnp.float32),
    )(be, bvalid, xs, w1, w3, w2)


def kernel(x, gate_w, w1, w2, w3):
    wts, idx = _router(x, gate_w)

    flat_e = idx.reshape(-1)
    eids = jnp.arange(NUM_EXPERTS, dtype=flat_e.dtype)
    oh = (flat_e[:, None] == eids[None, :]).astype(jnp.int32)
    cum = jnp.cumsum(oh, axis=0)
    rank = jnp.sum((cum - oh) * oh, axis=1)
    counts = cum[-1]
    pcounts = ((counts + BM - 1) // BM) * BM
    pstart = jnp.concatenate([jnp.zeros((1,), pcounts.dtype),
                              jnp.cumsum(pcounts)[:-1]])
    pos = (jnp.sum(pstart[None, :] * oh, axis=1) + rank).astype(jnp.int32)
    posr = pos.reshape(TOKENS, TOP_K)
    pos0, pos1 = posr[:, 0], posr[:, 1]

    bstart = jnp.arange(NB, dtype=jnp.int32) * BM
    le = (pstart[None, :].astype(jnp.int32) <= bstart[:, None])
    be = jnp.clip(jnp.sum(le.astype(jnp.int32), axis=1) - 1,
                  0, NUM_EXPERTS - 1).astype(jnp.int32)
    ohb = (be[:, None] == eids[None, :].astype(jnp.int32)).astype(jnp.int32)
    pstart_b = jnp.sum(pstart[None, :].astype(jnp.int32) * ohb, axis=1)
    counts_b = jnp.sum(counts[None, :].astype(jnp.int32) * ohb, axis=1)
    bvalid = ((bstart - pstart_b) < counts_b).astype(jnp.int32)

    w0x = jnp.broadcast_to(wts[:, 0:1], (TOKENS, 16))
    w1x = jnp.broadcast_to(wts[:, 1:2], (TOKENS, 16))

    xs = _sc_dispatch(x, pos0, pos1)
    ys = _grouped_ffn(xs, w1, w3, w2, be, bvalid)
    out = _sc_combine(ys, pos0, pos1, w0x, w1x)
    return out


# speedup vs baseline: 1.2874x; 1.2874x over previous
import functools

import jax
import jax.numpy as jnp
from jax import lax
from jax.experimental import pallas as pl
from jax.experimental.pallas import tpu as pltpu
from jax.experimental.pallas import tpu_sc as plsc

NUM_EXPERTS = 8
TOP_K = 2
HIDDEN = 1024
INTER = 4096
TOKENS = 2048

ASSIGN = TOKENS * TOP_K
BM = 256
NB = ASSIGN // BM + NUM_EXPERTS
P = NB * BM
BI = 1024
NI = INTER // BI

_NC = 2
_NS = 16
_NW = _NC * _NS

_NEG = -1e30


def _router_body(x_ref, gw_ref, wts_ref, idx_ref):
    x = x_ref[...]
    gw = gw_ref[...]
    logits = lax.dot_general(x, gw, (((1,), (1,)), ((), ())),
                             preferred_element_type=jnp.float32)
    col = lax.broadcasted_iota(jnp.int32, logits.shape, 1)
    logits = jnp.where(col < NUM_EXPERTS, logits, _NEG)
    m1 = jnp.max(logits, axis=1, keepdims=True)
    i1 = jnp.min(jnp.where(logits == m1, col, 128), axis=1, keepdims=True)
    l2 = jnp.where(col == i1, _NEG, logits)
    m2 = jnp.max(l2, axis=1, keepdims=True)
    i2 = jnp.min(jnp.where(l2 == m2, col, 128), axis=1, keepdims=True)
    wa = 1.0 / (1.0 + jnp.exp(m2 - m1))
    wb = 1.0 - wa
    wts_ref[...] = jnp.where(col == 0, wa, 0.0) + jnp.where(col == 1, wb, 0.0)
    idx_ref[...] = jnp.where(col == 0, i1, 0) + jnp.where(col == 1, i2, 0)


def _router(x, gate_w):
    gw_pad = jnp.zeros((128, HIDDEN), jnp.float32).at[:NUM_EXPERTS].set(gate_w)
    wts128, idx128 = pl.pallas_call(
        _router_body,
        out_shape=(jax.ShapeDtypeStruct((TOKENS, 128), jnp.float32),
                   jax.ShapeDtypeStruct((TOKENS, 128), jnp.int32)),
    )(x, gw_pad)
    return wts128[:, :TOP_K], idx128[:, :TOP_K]


_T2 = TOKENS // _NW


@functools.lru_cache(maxsize=None)
def _sc_dispatch_kernel():
    mesh = plsc.VectorSubcoreMesh(core_axis_name="c", subcore_axis_name="s",
                                  num_cores=_NC, num_subcores=_NS)

    @functools.partial(
        pl.kernel, mesh=mesh,
        out_type=jax.ShapeDtypeStruct((P, HIDDEN), jnp.float32),
        scratch_types=[
            pltpu.VMEM((_T2,), jnp.int32),
            pltpu.VMEM((_T2,), jnp.int32),
            pltpu.VMEM((_T2, HIDDEN), jnp.float32),
            pltpu.SemaphoreType.DMA,
            pltpu.SemaphoreType.DMA,
            pltpu.SemaphoreType.DMA,
        ],
    )
    def k(x_hbm, p0_hbm, p1_hbm, xs_hbm, i0, i1, buf, sem0, sem1, sem2):
        wid = lax.axis_index("s") * _NC + lax.axis_index("c")
        base = wid * _T2
        cx = pltpu.async_copy(x_hbm.at[pl.ds(base, _T2)], buf, sem0)
        pltpu.sync_copy(p0_hbm.at[pl.ds(base, _T2)], i0)
        pltpu.sync_copy(p1_hbm.at[pl.ds(base, _T2)], i1)
        cx.wait()
        ca = pltpu.async_copy(buf, xs_hbm.at[i0], sem1)
        cb = pltpu.async_copy(buf, xs_hbm.at[i1], sem2)
        ca.wait()
        cb.wait()

    return k


def _sc_dispatch(x, pos0, pos1):
    return _sc_dispatch_kernel()(x, pos0, pos1)


_T4 = TOKENS // _NW
_G4 = 16


@functools.lru_cache(maxsize=None)
def _sc_combine_kernel():
    mesh = plsc.VectorSubcoreMesh(core_axis_name="c", subcore_axis_name="s",
                                  num_cores=_NC, num_subcores=_NS)

    @functools.partial(
        pl.kernel, mesh=mesh,
        out_type=jax.ShapeDtypeStruct((TOKENS, HIDDEN), jnp.float32),
        scratch_types=[
            pltpu.VMEM((_T4,), jnp.int32),
            pltpu.VMEM((_T4,), jnp.int32),
            pltpu.VMEM((_T4, 16), jnp.float32),
            pltpu.VMEM((_T4, 16), jnp.float32),
            pltpu.VMEM((2 * _G4, HIDDEN), jnp.float32),
            pltpu.VMEM((2 * _G4, HIDDEN), jnp.float32),
            pltpu.SemaphoreType.DMA,
            pltpu.SemaphoreType.DMA,
        ],
    )
    def k(ys_hbm, p0_hbm, p1_hbm, w0_hbm, w1_hbm, out_hbm,
          i0, i1, bw0, bw1, bufa, bufb, sema, semb):
        wid = lax.axis_index("s") * _NC + lax.axis_index("c")
        base = wid * _T4
        pltpu.sync_copy(p0_hbm.at[pl.ds(base, _T4)], i0)
        pltpu.sync_copy(p1_hbm.at[pl.ds(base, _T4)], i1)
        pltpu.sync_copy(w0_hbm.at[pl.ds(base, _T4)], bw0)
        pltpu.sync_copy(w1_hbm.at[pl.ds(base, _T4)], bw1)
        ng = _T4 // _G4
        halfa = (bufa.at[pl.ds(0, _G4)], bufa.at[pl.ds(_G4, _G4)])
        halfb = (bufb.at[pl.ds(0, _G4)], bufb.at[pl.ds(_G4, _G4)])

        def issue(g):
            return (
                pltpu.async_copy(
                    ys_hbm.at[i0.at[pl.ds(g * _G4, _G4)]], halfa[g % 2], sema),
                pltpu.async_copy(
                    ys_hbm.at[i1.at[pl.ds(g * _G4, _G4)]], halfb[g % 2], semb),
            )

        copies = [issue(0), issue(1)]
        for g in range(ng):
            ca, cb = copies[g]
            ca.wait()
            cb.wait()
            ba, bb = halfa[g % 2], halfb[g % 2]

            def add_row(r, _, g=g, ba=ba, bb=bb):
                wa = bw0[g * _G4 + r, pl.ds(0, 16)]
                wb = bw1[g * _G4 + r, pl.ds(0, 16)]
                for c in range(HIDDEN // 16):
                    sl = pl.ds(c * 16, 16)
                    ba[r, sl] = ba[r, sl] * wa + bb[r, sl] * wb
                return _

            lax.fori_loop(0, _G4, add_row, 0)
            pltpu.sync_copy(ba, out_hbm.at[pl.ds(base + g * _G4, _G4)])
            if g + 2 < ng:
                copies.append(issue(g + 2))

    return k


def _sc_combine(ys, pos0, pos1, w0x, w1x):
    return _sc_combine_kernel()(ys, pos0, pos1, w0x, w1x)


def _gffn_body(be_ref, bv_ref, xi_ref, xs_ref, w1_ref, w3_ref, w2_ref, ys_ref):
    j = pl.program_id(0)
    i = pl.program_id(1)

    @pl.when(bv_ref[i] == 1)
    def _():
        xs = xs_ref[...]
        a1 = lax.dot_general(xs, w1_ref[0], (((1,), (1,)), ((), ())),
                             preferred_element_type=jnp.float32)
        a3 = lax.dot_general(xs, w3_ref[0], (((1,), (1,)), ((), ())),
                             preferred_element_type=jnp.float32)
        h = a1 * jax.nn.sigmoid(a1) * a3
        part = lax.dot_general(h, w2_ref[0], (((1,), (1,)), ((), ())),
                               preferred_element_type=jnp.float32)
        rows = pl.ds(i * BM, BM)

        @pl.when(j == 0)
        def _init():
            ys_ref[rows, :] = part

        @pl.when(j > 0)
        def _acc():
            ys_ref[rows, :] += part


def _grouped_ffn(xs, w1, w3, w2, be, bvalid, xvi):
    grid_spec = pltpu.PrefetchScalarGridSpec(
        num_scalar_prefetch=3,
        grid=(NI, NB),
        in_specs=[
            pl.BlockSpec((BM, HIDDEN), lambda j, i, be, bv, xi: (xi[i], 0)),
            pl.BlockSpec((1, BI, HIDDEN),
                         lambda j, i, be, bv, xi: (be[i], j, 0)),
            pl.BlockSpec((1, BI, HIDDEN),
                         lambda j, i, be, bv, xi: (be[i], j, 0)),
            pl.BlockSpec((1, HIDDEN, BI),
                         lambda j, i, be, bv, xi: (be[i], 0, j)),
        ],
        out_specs=pl.BlockSpec((P, HIDDEN), lambda j, i, be, bv, xi: (0, 0)),
    )
    return pl.pallas_call(
        _gffn_body,
        grid_spec=grid_spec,
        out_shape=jax.ShapeDtypeStruct((P, HIDDEN), jnp.float32),
    )(be, bvalid, xvi, xs, w1, w3, w2)


def kernel(x, gate_w, w1, w2, w3):
    wts, idx = _router(x, gate_w)

    flat_e = idx.reshape(-1)
    eids = jnp.arange(NUM_EXPERTS, dtype=flat_e.dtype)
    oh = (flat_e[:, None] == eids[None, :]).astype(jnp.int32)
    cum = jnp.cumsum(oh, axis=0)
    rank = jnp.sum((cum - oh) * oh, axis=1)
    counts = cum[-1]
    pcounts = ((counts + BM - 1) // BM) * BM
    pstart = jnp.concatenate([jnp.zeros((1,), pcounts.dtype),
                              jnp.cumsum(pcounts)[:-1]])
    pos = (jnp.sum(pstart[None, :] * oh, axis=1) + rank).astype(jnp.int32)
    posr = pos.reshape(TOKENS, TOP_K)
    pos0, pos1 = posr[:, 0], posr[:, 1]

    bstart = jnp.arange(NB, dtype=jnp.int32) * BM
    le = (pstart[None, :].astype(jnp.int32) <= bstart[:, None])
    be = jnp.clip(jnp.sum(le.astype(jnp.int32), axis=1) - 1,
                  0, NUM_EXPERTS - 1).astype(jnp.int32)
    ohb = (be[:, None] == eids[None, :].astype(jnp.int32)).astype(jnp.int32)
    pstart_b = jnp.sum(pstart[None, :].astype(jnp.int32) * ohb, axis=1)
    counts_b = jnp.sum(counts[None, :].astype(jnp.int32) * ohb, axis=1)
    bvalid = ((bstart - pstart_b) < counts_b).astype(jnp.int32)

    w0x = jnp.broadcast_to(wts[:, 0:1], (TOKENS, 16))
    w1x = jnp.broadcast_to(wts[:, 1:2], (TOKENS, 16))

    xs = _sc_dispatch(x, pos0, pos1)
    xvi = jnp.maximum(lax.cummax(
        jnp.where(bvalid == 1, jnp.arange(NB, dtype=jnp.int32), -1), axis=0), 0)
    ys = _grouped_ffn(xs, w1, w3, w2, be, bvalid, xvi)
    out = _sc_combine(ys, pos0, pos1, w0x, w1x)
    return out


# speedup vs baseline: 1.4742x; 1.1450x over previous
import functools

import jax
import jax.numpy as jnp
from jax import lax
from jax.experimental import pallas as pl
from jax.experimental.pallas import tpu as pltpu
from jax.experimental.pallas import tpu_sc as plsc

NUM_EXPERTS = 8
TOP_K = 2
HIDDEN = 1024
INTER = 4096
TOKENS = 2048

ASSIGN = TOKENS * TOP_K
BM = 512
NB = ASSIGN // BM + NUM_EXPERTS
P = NB * BM
BI = 1024
NI = INTER // BI

_NC = 2
_NS = 16
_NW = _NC * _NS

_NEG = -1e30


def _router_body(x_ref, gw_ref, wts_ref, idx_ref):
    x = x_ref[...]
    gw = gw_ref[...]
    logits = lax.dot_general(x, gw, (((1,), (1,)), ((), ())),
                             preferred_element_type=jnp.float32)
    col = lax.broadcasted_iota(jnp.int32, logits.shape, 1)
    logits = jnp.where(col < NUM_EXPERTS, logits, _NEG)
    m1 = jnp.max(logits, axis=1, keepdims=True)
    i1 = jnp.min(jnp.where(logits == m1, col, 128), axis=1, keepdims=True)
    l2 = jnp.where(col == i1, _NEG, logits)
    m2 = jnp.max(l2, axis=1, keepdims=True)
    i2 = jnp.min(jnp.where(l2 == m2, col, 128), axis=1, keepdims=True)
    wa = 1.0 / (1.0 + jnp.exp(m2 - m1))
    wb = 1.0 - wa
    wts_ref[...] = jnp.where(col == 0, wa, 0.0) + jnp.where(col == 1, wb, 0.0)
    idx_ref[...] = jnp.where(col == 0, i1, 0) + jnp.where(col == 1, i2, 0)


def _router(x, gate_w):
    gw_pad = jnp.zeros((128, HIDDEN), jnp.float32).at[:NUM_EXPERTS].set(gate_w)
    wts128, idx128 = pl.pallas_call(
        _router_body,
        out_shape=(jax.ShapeDtypeStruct((TOKENS, 128), jnp.float32),
                   jax.ShapeDtypeStruct((TOKENS, 128), jnp.int32)),
    )(x, gw_pad)
    return wts128[:, :TOP_K], idx128[:, :TOP_K]


_T2 = TOKENS // _NW


@functools.lru_cache(maxsize=None)
def _sc_dispatch_kernel():
    mesh = plsc.VectorSubcoreMesh(core_axis_name="c", subcore_axis_name="s",
                                  num_cores=_NC, num_subcores=_NS)

    @functools.partial(
        pl.kernel, mesh=mesh,
        out_type=jax.ShapeDtypeStruct((P, HIDDEN), jnp.float32),
        scratch_types=[
            pltpu.VMEM((_T2,), jnp.int32),
            pltpu.VMEM((_T2,), jnp.int32),
            pltpu.VMEM((_T2, HIDDEN), jnp.float32),
            pltpu.SemaphoreType.DMA,
            pltpu.SemaphoreType.DMA,
            pltpu.SemaphoreType.DMA,
        ],
    )
    def k(x_hbm, p0_hbm, p1_hbm, xs_hbm, i0, i1, buf, sem0, sem1, sem2):
        wid = lax.axis_index("s") * _NC + lax.axis_index("c")
        base = wid * _T2
        cx = pltpu.async_copy(x_hbm.at[pl.ds(base, _T2)], buf, sem0)
        pltpu.sync_copy(p0_hbm.at[pl.ds(base, _T2)], i0)
        pltpu.sync_copy(p1_hbm.at[pl.ds(base, _T2)], i1)
        cx.wait()
        ca = pltpu.async_copy(buf, xs_hbm.at[i0], sem1)
        cb = pltpu.async_copy(buf, xs_hbm.at[i1], sem2)
        ca.wait()
        cb.wait()

    return k


def _sc_dispatch(x, pos0, pos1):
    return _sc_dispatch_kernel()(x, pos0, pos1)


_T4 = TOKENS // _NW
_G4 = 16


@functools.lru_cache(maxsize=None)
def _sc_combine_kernel():
    mesh = plsc.VectorSubcoreMesh(core_axis_name="c", subcore_axis_name="s",
                                  num_cores=_NC, num_subcores=_NS)

    @functools.partial(
        pl.kernel, mesh=mesh,
        out_type=jax.ShapeDtypeStruct((TOKENS, HIDDEN), jnp.float32),
        scratch_types=[
            pltpu.VMEM((_T4,), jnp.int32),
            pltpu.VMEM((_T4,), jnp.int32),
            pltpu.VMEM((_T4, 16), jnp.float32),
            pltpu.VMEM((_T4, 16), jnp.float32),
            pltpu.VMEM((2 * _G4, HIDDEN), jnp.float32),
            pltpu.VMEM((2 * _G4, HIDDEN), jnp.float32),
            pltpu.SemaphoreType.DMA,
            pltpu.SemaphoreType.DMA,
        ],
    )
    def k(ys_hbm, p0_hbm, p1_hbm, w0_hbm, w1_hbm, out_hbm,
          i0, i1, bw0, bw1, bufa, bufb, sema, semb):
        wid = lax.axis_index("s") * _NC + lax.axis_index("c")
        base = wid * _T4
        pltpu.sync_copy(p0_hbm.at[pl.ds(base, _T4)], i0)
        pltpu.sync_copy(p1_hbm.at[pl.ds(base, _T4)], i1)
        pltpu.sync_copy(w0_hbm.at[pl.ds(base, _T4)], bw0)
        pltpu.sync_copy(w1_hbm.at[pl.ds(base, _T4)], bw1)
        ng = _T4 // _G4
        halfa = (bufa.at[pl.ds(0, _G4)], bufa.at[pl.ds(_G4, _G4)])
        halfb = (bufb.at[pl.ds(0, _G4)], bufb.at[pl.ds(_G4, _G4)])

        def issue(g):
            return (
                pltpu.async_copy(
                    ys_hbm.at[i0.at[pl.ds(g * _G4, _G4)]], halfa[g % 2], sema),
                pltpu.async_copy(
                    ys_hbm.at[i1.at[pl.ds(g * _G4, _G4)]], halfb[g % 2], semb),
            )

        copies = [issue(0), issue(1)]
        for g in range(ng):
            ca, cb = copies[g]
            ca.wait()
            cb.wait()
            ba, bb = halfa[g % 2], halfb[g % 2]

            def add_row(r, _, g=g, ba=ba, bb=bb):
                wa = bw0[g * _G4 + r, pl.ds(0, 16)]
                wb = bw1[g * _G4 + r, pl.ds(0, 16)]
                for c in range(HIDDEN // 16):
                    sl = pl.ds(c * 16, 16)
                    ba[r, sl] = ba[r, sl] * wa + bb[r, sl] * wb
                return _

            lax.fori_loop(0, _G4, add_row, 0)
            pltpu.sync_copy(ba, out_hbm.at[pl.ds(base + g * _G4, _G4)])
            if g + 2 < ng:
                copies.append(issue(g + 2))

    return k


def _sc_combine(ys, pos0, pos1, w0x, w1x):
    return _sc_combine_kernel()(ys, pos0, pos1, w0x, w1x)


def _gffn_body(be_ref, bv_ref, xi_ref, xs_ref, w1_ref, w3_ref, w2_ref, ys_ref):
    j = pl.program_id(0)
    i = pl.program_id(1)

    @pl.when(bv_ref[i] == 1)
    def _():
        xs = xs_ref[...]
        a1 = lax.dot_general(xs, w1_ref[0], (((1,), (1,)), ((), ())),
                             preferred_element_type=jnp.float32)
        a3 = lax.dot_general(xs, w3_ref[0], (((1,), (1,)), ((), ())),
                             preferred_element_type=jnp.float32)
        h = a1 * jax.nn.sigmoid(a1) * a3
        part = lax.dot_general(h, w2_ref[0], (((1,), (1,)), ((), ())),
                               preferred_element_type=jnp.float32)
        rows = pl.ds(i * BM, BM)

        @pl.when(j == 0)
        def _init():
            ys_ref[rows, :] = part

        @pl.when(j > 0)
        def _acc():
            ys_ref[rows, :] += part


def _grouped_ffn(xs, w1, w3, w2, be, bvalid, xvi):
    grid_spec = pltpu.PrefetchScalarGridSpec(
        num_scalar_prefetch=3,
        grid=(NI, NB),
        in_specs=[
            pl.BlockSpec((BM, HIDDEN), lambda j, i, be, bv, xi: (xi[i], 0)),
            pl.BlockSpec((1, BI, HIDDEN),
                         lambda j, i, be, bv, xi: (be[i], j, 0)),
            pl.BlockSpec((1, BI, HIDDEN),
                         lambda j, i, be, bv, xi: (be[i], j, 0)),
            pl.BlockSpec((1, HIDDEN, BI),
                         lambda j, i, be, bv, xi: (be[i], 0, j)),
        ],
        out_specs=pl.BlockSpec((P, HIDDEN), lambda j, i, be, bv, xi: (0, 0)),
    )
    return pl.pallas_call(
        _gffn_body,
        grid_spec=grid_spec,
        out_shape=jax.ShapeDtypeStruct((P, HIDDEN), jnp.float32),
        compiler_params=pltpu.CompilerParams(
            vmem_limit_bytes=100 * 1024 * 1024),
    )(be, bvalid, xvi, xs, w1, w3, w2)


def kernel(x, gate_w, w1, w2, w3):
    wts, idx = _router(x, gate_w)

    flat_e = idx.reshape(-1)
    eids = jnp.arange(NUM_EXPERTS, dtype=flat_e.dtype)
    oh = (flat_e[:, None] == eids[None, :]).astype(jnp.int32)
    cum = jnp.cumsum(oh, axis=0)
    rank = jnp.sum((cum - oh) * oh, axis=1)
    counts = cum[-1]
    pcounts = ((counts + BM - 1) // BM) * BM
    pstart = jnp.concatenate([jnp.zeros((1,), pcounts.dtype),
                              jnp.cumsum(pcounts)[:-1]])
    pos = (jnp.sum(pstart[None, :] * oh, axis=1) + rank).astype(jnp.int32)
    posr = pos.reshape(TOKENS, TOP_K)
    pos0, pos1 = posr[:, 0], posr[:, 1]

    bstart = jnp.arange(NB, dtype=jnp.int32) * BM
    le = (pstart[None, :].astype(jnp.int32) <= bstart[:, None])
    be = jnp.clip(jnp.sum(le.astype(jnp.int32), axis=1) - 1,
                  0, NUM_EXPERTS - 1).astype(jnp.int32)
    ohb = (be[:, None] == eids[None, :].astype(jnp.int32)).astype(jnp.int32)
    pstart_b = jnp.sum(pstart[None, :].astype(jnp.int32) * ohb, axis=1)
    counts_b = jnp.sum(counts[None, :].astype(jnp.int32) * ohb, axis=1)
    bvalid = ((bstart - pstart_b) < counts_b).astype(jnp.int32)

    w0x = jnp.broadcast_to(wts[:, 0:1], (TOKENS, 16))
    w1x = jnp.broadcast_to(wts[:, 1:2], (TOKENS, 16))

    xs = _sc_dispatch(x, pos0, pos1)
    xvi = jnp.maximum(lax.cummax(
        jnp.where(bvalid == 1, jnp.arange(NB, dtype=jnp.int32), -1), axis=0), 0)
    ys = _grouped_ffn(xs, w1, w3, w2, be, bvalid, xvi)
    out = _sc_combine(ys, pos0, pos1, w0x, w1x)
    return out
